# baseline (speedup 1.0000x reference)
import functools

import jax
import jax.numpy as jnp
from jax import lax
from jax.experimental import pallas as pl
from jax.experimental.pallas import tpu as pltpu
from jax.experimental.pallas import tpu_sc as plsc


def _bow_kernel(B, H, V, D):
    info = plsc.get_sparse_core_info()
    NC, NS, L = info.num_cores, info.num_subcores, info.num_lanes
    NW = NC * NS
    assert B % NW == 0 and D % L == 0
    bpw = B // NW
    NBUF = 8
    assert bpw % NBUF == 0

    mesh = plsc.VectorSubcoreMesh(core_axis_name="c", subcore_axis_name="s")

    @functools.partial(
        pl.kernel,
        mesh=mesh,
        out_type=jax.ShapeDtypeStruct((B, D), jnp.float32),
        scratch_types=[
            pltpu.VMEM((bpw, H), jnp.int32),
            pltpu.VMEM((NBUF, H, D), jnp.float32),
            pltpu.VMEM((bpw, D), jnp.float32),
            [pltpu.SemaphoreType.DMA] * NBUF,
        ],
        compiler_params=pltpu.CompilerParams(use_tc_tiling_on_sc=False),
    )
    def body(x_hbm, tab_hbm, out_hbm, idx_v, rows_v, out_v, sems):
        wid = lax.axis_index("s") * NC + lax.axis_index("c")
        base = wid * bpw
        pltpu.sync_copy(x_hbm.at[pl.ds(base, bpw), :], idx_v)

        inv_h = jnp.float32(1.0 / H)

        for k in range(NBUF):
            pltpu.async_copy(tab_hbm.at[idx_v.at[k]], rows_v.at[k], sems[k])

        def step(g, carry):
            for k in range(NBUF):
                b = g * NBUF + k
                pltpu.make_async_copy(
                    tab_hbm.at[idx_v.at[0]], rows_v.at[k], sems[k]
                ).wait()
                rows = rows_v.at[k]
                for d in range(D // L):
                    sl = pl.ds(d * L, L)
                    acc = rows[0, sl]
                    for j in range(1, H):
                        acc = acc + rows[j, sl]
                    out_v[b, sl] = acc * inv_h
                nb = jnp.minimum(b + NBUF, bpw - 1)
                pltpu.async_copy(tab_hbm.at[idx_v.at[nb]], rows_v.at[k], sems[k])

            return carry

        lax.fori_loop(0, bpw // NBUF, step, 0)

        for k in range(NBUF):
            pltpu.make_async_copy(
                tab_hbm.at[idx_v.at[0]], rows_v.at[k], sems[k]
            ).wait()

        pltpu.sync_copy(out_v, out_hbm.at[pl.ds(base, bpw), :])

    return body


def kernel(x, table):
    B, H = x.shape
    V, D = table.shape
    x = x.astype(jnp.int32)
    return _bow_kernel(B, H, V, D)(x, table)


# speedup vs baseline: 1.0596x; 1.0596x over previous
import functools

import jax
import jax.numpy as jnp
from jax import lax
from jax.experimental import pallas as pl
from jax.experimental.pallas import tpu as pltpu
from jax.experimental.pallas import tpu_sc as plsc


def _bow_kernel(B, H, V, D):
    info = plsc.get_sparse_core_info()
    NC, NS, L = info.num_cores, info.num_subcores, info.num_lanes
    NW = NC * NS
    assert B % NW == 0 and D % L == 0 and V % 2 == 0
    bpw = B // NW
    NBUF = 4
    assert bpw % NBUF == 0

    mesh = plsc.VectorSubcoreMesh(core_axis_name="c", subcore_axis_name="s")

    @functools.partial(
        pl.kernel,
        mesh=mesh,
        out_type=jax.ShapeDtypeStruct((B, D), jnp.float32),
        scratch_types=[
            pltpu.VMEM((bpw, H), jnp.int32),
            pltpu.VMEM((bpw, H), jnp.int32),
            pltpu.VMEM((NBUF, H, 2 * D), jnp.float32),
            pltpu.VMEM((bpw, D), jnp.float32),
            [pltpu.SemaphoreType.DMA] * NBUF,
        ],
        compiler_params=pltpu.CompilerParams(use_tc_tiling_on_sc=True),
    )
    def body(x_hbm, xp_hbm, tab_hbm, out_hbm, idx_v, idxp_v, rows_v, out_v, sems):
        wid = lax.axis_index("s") * NC + lax.axis_index("c")
        base = wid * bpw
        pltpu.sync_copy(x_hbm.at[pl.ds(base, bpw), :], idx_v)
        pltpu.sync_copy(xp_hbm.at[pl.ds(base, bpw), :], idxp_v)

        inv_h = jnp.float32(1.0 / H)

        for k in range(NBUF):
            pltpu.async_copy(tab_hbm.at[idxp_v.at[k]], rows_v.at[k], sems[k])

        def step(g, carry):
            for k in range(NBUF):
                b = g * NBUF + k
                pltpu.make_async_copy(
                    tab_hbm.at[idxp_v.at[0]], rows_v.at[k], sems[k]
                ).wait()
                rows = rows_v.at[k]
                starts = []
                s = 0
                while s + L < H:
                    starts.append(s)
                    s += L
                starts.append(H - L)
                offv = [
                    (idx_v[b, pl.ds(s0, L)] & 1) * D for s0 in starts
                ]
                accs = [None] * (D // L)
                for j in range(H):
                    w = min(j // L, len(starts) - 1)
                    off = offv[w][j - starts[w]]
                    for d in range(D // L):
                        val = rows[j, pl.ds(off + d * L, L)]
                        accs[d] = val if accs[d] is None else accs[d] + val
                for d in range(D // L):
                    out_v[b, pl.ds(d * L, L)] = accs[d] * inv_h
                nb = jnp.minimum(b + NBUF, bpw - 1)
                pltpu.async_copy(tab_hbm.at[idxp_v.at[nb]], rows_v.at[k], sems[k])

            return carry

        lax.fori_loop(0, bpw // NBUF, step, 0)

        for k in range(NBUF):
            pltpu.make_async_copy(
                tab_hbm.at[idxp_v.at[0]], rows_v.at[k], sems[k]
            ).wait()

        pltpu.sync_copy(out_v, out_hbm.at[pl.ds(base, bpw), :])

    return body


def kernel(x, table):
    B, H = x.shape
    V, D = table.shape
    x = x.astype(jnp.int32)
    xp = jax.lax.shift_right_logical(x, 1)
    tab_pairs = table.reshape(V // 2, 2 * D)
    return _bow_kernel(B, H, V, D)(x, xp, tab_pairs)


# speedup vs baseline: 1.0632x; 1.0034x over previous
import functools

import jax
import jax.numpy as jnp
from jax import lax
from jax.experimental import pallas as pl
from jax.experimental.pallas import tpu as pltpu
from jax.experimental.pallas import tpu_sc as plsc


def _bow_kernel(B, H, V, D):
    info = plsc.get_sparse_core_info()
    NC, NS, L = info.num_cores, info.num_subcores, info.num_lanes
    NW = NC * NS
    assert B % NW == 0 and D % L == 0 and V % 2 == 0
    bpw = B // NW
    NBUF = 4
    assert bpw % NBUF == 0
    PADW = 2 * D

    mesh = plsc.VectorSubcoreMesh(core_axis_name="c", subcore_axis_name="s")

    @functools.partial(
        pl.kernel,
        mesh=mesh,
        out_type=jax.ShapeDtypeStruct((B, D), jnp.float32),
        scratch_types=[
            pltpu.VMEM((bpw, H), jnp.int32),
            pltpu.VMEM((bpw, H), jnp.int32),
            pltpu.VMEM((NBUF, H, 2 * D), jnp.float32),
            pltpu.VMEM((bpw, D), jnp.float32),
            [pltpu.SemaphoreType.DMA] * NBUF,
        ],
        compiler_params=pltpu.CompilerParams(use_tc_tiling_on_sc=True),
    )
    def body(x_hbm, tabv, out_hbm, idx_v, idxp_v, rows_v, out_v, sems):
        wid = lax.axis_index("s") * NC + lax.axis_index("c")
        base = wid * bpw
        pltpu.sync_copy(x_hbm.at[pl.ds(base, bpw), :], idx_v)

        inv_h = jnp.float32(1.0 / H)

        starts = []
        s = 0
        while s + L < H:
            starts.append(s)
            s += L
        starts.append(H - L)

        def fill_pair_row(b):
            for s0 in starts:
                sl = pl.ds(s0, L)
                idxp_v[b, sl] = lax.shift_right_logical(idx_v[b, sl], 1)

        for k in range(NBUF):
            fill_pair_row(k)
            pltpu.async_copy(tabv.at[idxp_v.at[k]], rows_v.at[k], sems[k])

        def step(g, carry):
            for k in range(NBUF):
                b = g * NBUF + k
                pltpu.make_async_copy(
                    tabv.at[idxp_v.at[0]], rows_v.at[k], sems[k]
                ).wait()
                rows = rows_v.at[k]
                offv = [(idx_v[b, pl.ds(s0, L)] & 1) * D for s0 in starts]
                accs = [None] * (D // L)
                for j in range(H):
                    w = min(j // L, len(starts) - 1)
                    off = offv[w][j - starts[w]]
                    for d in range(D // L):
                        val = rows[j, pl.ds(off + d * L, L)]
                        accs[d] = val if accs[d] is None else accs[d] + val
                for d in range(D // L):
                    out_v[b, pl.ds(d * L, L)] = accs[d] * inv_h
                nb = jnp.minimum(b + NBUF, bpw - 1)
                fill_pair_row(nb)
                pltpu.async_copy(tabv.at[idxp_v.at[nb]], rows_v.at[k], sems[k])

            return carry

        lax.fori_loop(0, bpw // NBUF, step, 0)

        for k in range(NBUF):
            pltpu.make_async_copy(
                tabv.at[idxp_v.at[0]], rows_v.at[k], sems[k]
            ).wait()

        pltpu.sync_copy(out_v, out_hbm.at[pl.ds(base, bpw), :])

    return body


def kernel(x, table):
    B, H = x.shape
    V, D = table.shape
    x = x.astype(jnp.int32)
    tab_pairs = table.reshape(V // 2, 2 * D)
    return _bow_kernel(B, H, V, D)(x, tab_pairs)
